# baseline (speedup 1.0000x reference)
import functools

import jax
import jax.numpy as jnp
from jax import lax
from jax.experimental import pallas as pl
from jax.experimental.pallas import tpu as pltpu
from jax.experimental.pallas import tpu_sc as plsc

E = 8
H = 768
F = 2048
TT = 512
NTP = 16
PR = NTP * TT
LB_W = 0.01


def _gate_body(x_ref, gw_ref, pos1_ref, pos2_ref, w1x_ref, w2x_ref,
               te_ref, send_ref, slot_ref, dnx_ref, aux_ref):
    x = x_ref[...]
    gw = gw_ref[...]
    s = x.shape[0]
    logits = jax.lax.dot_general(
        x, gw, (((1,), (1,)), ((), ())), preferred_element_type=jnp.float32)
    m = jnp.max(logits, axis=-1, keepdims=True)
    ex = jnp.exp(logits - m)
    probs = ex / jnp.sum(ex, axis=-1, keepdims=True)

    iota = jax.lax.broadcasted_iota(jnp.int32, probs.shape, 1)
    p1 = jnp.max(probs, axis=-1, keepdims=True)
    idx1 = jnp.min(jnp.where(probs == p1, iota, E), axis=-1, keepdims=True)
    oh1 = (iota == idx1)
    masked = jnp.where(oh1, -jnp.inf, probs)
    p2 = jnp.max(masked, axis=-1, keepdims=True)
    idx2 = jnp.min(jnp.where(masked == p2, iota, E), axis=-1, keepdims=True)
    oh2 = (iota == idx2)

    denom = p1 + p2 + 1e-9
    oh1f = oh1.astype(jnp.float32)
    oh2f = oh2.astype(jnp.float32)
    w1x_ref[...] = jnp.broadcast_to(p1 / denom, (s, 128))
    w2x_ref[...] = jnp.broadcast_to(p2 / denom, (s, 128))

    sf = jnp.float32(s)
    f = jnp.sum(oh1f + oh2f, axis=0) / sf
    pmean = jnp.sum(probs, axis=0) / sf
    aux_ref[...] = jnp.reshape(LB_W * E * jnp.sum(f * pmean), (1, 1))

    O = jnp.concatenate([oh1f, oh2f], axis=0)
    a2 = 2 * s
    ri = jax.lax.broadcasted_iota(jnp.int32, (a2, a2), 0)
    ci = jax.lax.broadcasted_iota(jnp.int32, (a2, a2), 1)
    L = (ci < ri).astype(jnp.bfloat16)
    rank = jax.lax.dot_general(
        L, O.astype(jnp.bfloat16), (((1,), (0,)), ((), ())),
        preferred_element_type=jnp.float32)

    counts = jnp.sum(O, axis=0, keepdims=True)
    pcounts = jnp.ceil(counts / TT) * TT
    ce = jax.lax.broadcasted_iota(jnp.int32, (E, E), 0)
    cc = jax.lax.broadcasted_iota(jnp.int32, (E, E), 1)
    tri = (ce < cc).astype(jnp.float32)
    offs_pad = jax.lax.dot_general(
        pcounts, tri, (((1,), (0,)), ((), ())),
        preferred_element_type=jnp.float32)

    pos = jnp.sum(O * (rank + offs_pad), axis=1, keepdims=True)
    posi = pos.astype(jnp.int32)
    pos1_ref[...] = posi[:s]
    pos2_ref[...] = posi[s:]

    tstart = (jax.lax.broadcasted_iota(jnp.int32, (1, NTP), 1) * TT
              ).astype(jnp.float32)
    op_col = jnp.broadcast_to(offs_pad.reshape(E, 1), (E, NTP))
    te = jnp.sum((op_col <= tstart).astype(jnp.float32), axis=0,
                 keepdims=True) - 1.0
    seg_end = offs_pad + counts
    te_b = jnp.broadcast_to(te, (E, NTP))
    e_col = jax.lax.broadcasted_iota(jnp.int32, (E, NTP), 0).astype(jnp.float32)
    send = jnp.sum(jnp.where(te_b == e_col,
                             jnp.broadcast_to(seg_end.reshape(E, 1), (E, NTP)),
                             0.0), axis=0, keepdims=True)
    te_ref[...] = te.astype(jnp.int32)
    send_ref[...] = send.astype(jnp.int32)

    pres = (counts > 0.0).astype(jnp.float32)
    ordv = jax.lax.dot_general(
        pres, tri, (((1,), (0,)), ((), ())),
        preferred_element_type=jnp.float32)
    ord_te = jnp.sum(jnp.where(te_b == e_col,
                               jnp.broadcast_to(ordv.reshape(E, 1), (E, NTP)),
                               0.0), axis=0, keepdims=True)
    slot_ref[...] = jnp.bitwise_and(ord_te.astype(jnp.int32), 1)

    tv_next = jnp.concatenate([te[:, 1:], te[:, NTP - 1:]], axis=1)
    val_col = jnp.broadcast_to(te.reshape(NTP, 1), (NTP, NTP))
    dmat = jnp.where(val_col > tv_next, val_col, 99.0)
    dnx_ref[...] = jnp.min(dmat, axis=0, keepdims=True).astype(jnp.int32)


def _ffn_body(te_ref, send_ref, slot_ref, dnx_ref, xg_ref, ws_ref,
              wgu_any, wd_any, y_ref, wgu_st, wd_st, wgub, wdb,
              sem_gu, sem_d):
    n = pl.program_id(0)
    e = te_ref[n]
    sl = slot_ref[n]
    seg_end = send_ref[n]
    n1 = jnp.minimum(n + 1, NTP - 1)
    t1 = te_ref[n1]
    s1 = slot_ref[n1]
    d = dnx_ref[n]

    def _issue(ex):
        pltpu.make_async_copy(wgu_any.at[ex], wgu_st, sem_gu).start()
        pltpu.make_async_copy(wd_any.at[ex], wd_st, sem_d).start()

    def _wait_cast(ex, bslot):
        pltpu.make_async_copy(wgu_any.at[ex], wgu_st, sem_gu).wait()
        pltpu.make_async_copy(wd_any.at[ex], wd_st, sem_d).wait()
        wgub[bslot] = wgu_st[...].astype(jnp.bfloat16)
        wdb[bslot] = wd_st[...].astype(jnp.bfloat16)

    @pl.when(n == 0)
    def _():
        _issue(e)
        _wait_cast(e, sl)
        tgt = jnp.where(t1 != e, t1, d)

        @pl.when(tgt < E)
        def _():
            _issue(tgt)

    used = seg_end > n * TT

    @pl.when(used)
    def _():
        ri = jax.lax.broadcasted_iota(jnp.int32, (TT, 1), 0) + n * TT
        rmask = ri < seg_end
        x = jnp.where(rmask, xg_ref[...], 0.0).astype(jnp.bfloat16)
        gu = jnp.dot(x, wgub[sl], preferred_element_type=jnp.float32)
        g = gu[:, :F]
        u = gu[:, F:]
        act = (g * jax.nn.sigmoid(g) * u).astype(jnp.bfloat16)
        y = jnp.dot(act, wdb[sl], preferred_element_type=jnp.float32)
        ws = jnp.where(rmask, ws_ref[:, 0:1], 0.0)
        y_ref[...] = y * ws

    @pl.when(jnp.logical_not(used))
    def _():
        y_ref[...] = jnp.zeros_like(y_ref)

    @pl.when(t1 != e)
    def _():
        _wait_cast(t1, s1)

        @pl.when(d < E)
        def _():
            _issue(d)


_info = plsc.get_sparse_core_info()
_NC = _info.num_cores
_NS = _info.num_subcores
_NW = _NC * _NS


def _make_scatter(s, h):
    tpw = s // _NW
    mesh = plsc.VectorSubcoreMesh(core_axis_name="c", subcore_axis_name="s")

    @functools.partial(
        pl.kernel, mesh=mesh,
        out_type=[
            jax.ShapeDtypeStruct((PR, h), jnp.float32),
            jax.ShapeDtypeStruct((PR, 128), jnp.float32),
        ],
        scratch_types=[
            pltpu.VMEM((tpw,), jnp.int32),
            pltpu.VMEM((tpw,), jnp.int32),
            pltpu.VMEM((tpw, h), jnp.float32),
            pltpu.VMEM((tpw, 128), jnp.float32),
            pltpu.VMEM((tpw, 128), jnp.float32),
            pltpu.SemaphoreType.DMA,
        ],
    )
    def k(x_hbm, pos1_hbm, pos2_hbm, w1x_hbm, w2x_hbm, xg_hbm, ws_hbm,
          idx1_v, idx2_v, xv, w1v, w2v, sem):
        wid = lax.axis_index("s") * _NC + lax.axis_index("c")
        base = wid * tpw
        pltpu.sync_copy(pos1_hbm.at[pl.ds(base, tpw)], idx1_v)
        pltpu.sync_copy(pos2_hbm.at[pl.ds(base, tpw)], idx2_v)
        pltpu.sync_copy(x_hbm.at[pl.ds(base, tpw), :], xv)
        pltpu.sync_copy(w1x_hbm.at[pl.ds(base, tpw), :], w1v)
        pltpu.sync_copy(w2x_hbm.at[pl.ds(base, tpw), :], w2v)
        c1 = pltpu.async_copy(xv, xg_hbm.at[idx1_v], sem)
        c2 = pltpu.async_copy(xv, xg_hbm.at[idx2_v], sem)
        c3 = pltpu.async_copy(w1v, ws_hbm.at[idx1_v], sem)
        c4 = pltpu.async_copy(w2v, ws_hbm.at[idx2_v], sem)
        c1.wait()
        c2.wait()
        c3.wait()
        c4.wait()

    return k


def _make_combine(s, h):
    tpw = s // _NW
    half = tpw // 2
    mesh = plsc.VectorSubcoreMesh(core_axis_name="c", subcore_axis_name="s")

    @functools.partial(
        pl.kernel, mesh=mesh,
        out_type=jax.ShapeDtypeStruct((s, h), jnp.float32),
        scratch_types=[
            pltpu.VMEM((half,), jnp.int32),
            pltpu.VMEM((half,), jnp.int32),
            pltpu.VMEM((half, h), jnp.float32),
            pltpu.VMEM((half, h), jnp.float32),
            pltpu.SemaphoreType.DMA,
        ],
    )
    def k(y_hbm, pos1_hbm, pos2_hbm, out_hbm, i1, i2, r1, r2, sem):
        wid = lax.axis_index("s") * _NC + lax.axis_index("c")

        def chunk(ci, carry):
            base = wid * tpw + ci * half
            pltpu.sync_copy(pos1_hbm.at[pl.ds(base, half)], i1)
            pltpu.sync_copy(pos2_hbm.at[pl.ds(base, half)], i2)
            pltpu.async_copy(y_hbm.at[i1], r1, sem).wait()
            pltpu.async_copy(y_hbm.at[i2], r2, sem).wait()

            def row(r, carry2):
                for c in range(h // 16):
                    sl = pl.ds(c * 16, 16)
                    r1[r, sl] = r1[r, sl] + r2[r, sl]
                return carry2

            lax.fori_loop(0, half, row, 0)
            pltpu.sync_copy(r1, out_hbm.at[pl.ds(base, half), :])
            return carry

        lax.fori_loop(0, 2, chunk, 0)

    return k


def kernel(x, gate_w, w_gate_up, w_down):
    b, s, h = x.shape
    x_flat = x.reshape(s, h)

    pos1, pos2, w1x, w2x, te, send, slot, dnx, aux = pl.pallas_call(
        _gate_body,
        out_shape=[
            jax.ShapeDtypeStruct((s, 1), jnp.int32),
            jax.ShapeDtypeStruct((s, 1), jnp.int32),
            jax.ShapeDtypeStruct((s, 128), jnp.float32),
            jax.ShapeDtypeStruct((s, 128), jnp.float32),
            jax.ShapeDtypeStruct((1, NTP), jnp.int32),
            jax.ShapeDtypeStruct((1, NTP), jnp.int32),
            jax.ShapeDtypeStruct((1, NTP), jnp.int32),
            jax.ShapeDtypeStruct((1, NTP), jnp.int32),
            jax.ShapeDtypeStruct((1, 1), jnp.float32),
        ],
        compiler_params=pltpu.CompilerParams(
            vmem_limit_bytes=120 * 1024 * 1024,
        ),
    )(x_flat, gate_w)

    p1 = pos1.reshape(s)
    p2 = pos2.reshape(s)
    xg, wsort = _make_scatter(s, h)(x_flat, p1, p2, w1x, w2x)

    grid_spec = pltpu.PrefetchScalarGridSpec(
        num_scalar_prefetch=4,
        grid=(NTP,),
        in_specs=[
            pl.BlockSpec((TT, H), lambda n, *_: (n, 0)),
            pl.BlockSpec((TT, 128), lambda n, *_: (n, 0)),
            pl.BlockSpec(memory_space=pl.ANY),
            pl.BlockSpec(memory_space=pl.ANY),
        ],
        out_specs=pl.BlockSpec((TT, H), lambda n, *_: (n, 0)),
        scratch_shapes=[
            pltpu.VMEM((H, 2 * F), jnp.float32),
            pltpu.VMEM((F, H), jnp.float32),
            pltpu.VMEM((2, H, 2 * F), jnp.bfloat16),
            pltpu.VMEM((2, F, H), jnp.bfloat16),
            pltpu.SemaphoreType.DMA,
            pltpu.SemaphoreType.DMA,
        ],
    )
    y = pl.pallas_call(
        _ffn_body,
        grid_spec=grid_spec,
        out_shape=jax.ShapeDtypeStruct((PR, H), jnp.float32),
        compiler_params=pltpu.CompilerParams(
            vmem_limit_bytes=120 * 1024 * 1024,
        ),
    )(te.reshape(NTP), send.reshape(NTP), slot.reshape(NTP),
      dnx.reshape(NTP), xg, wsort, w_gate_up, w_down)

    out = _make_combine(s, h)(y, p1, p2)
    return out.reshape(b, s, h), aux[0, 0]


# speedup vs baseline: 1.0030x; 1.0030x over previous
import functools

import jax
import jax.numpy as jnp
from jax import lax
from jax.experimental import pallas as pl
from jax.experimental.pallas import tpu as pltpu
from jax.experimental.pallas import tpu_sc as plsc

E = 8
H = 768
F = 2048
TT = 768
NTP = 14
PR = NTP * TT
LB_W = 0.01


def _gate_body(x_ref, gw_ref, pos1_ref, pos2_ref, w1x_ref, w2x_ref,
               te_ref, send_ref, aux_ref):
    x = x_ref[...]
    gw = gw_ref[...]
    s = x.shape[0]
    logits = jax.lax.dot_general(
        x, gw, (((1,), (1,)), ((), ())), preferred_element_type=jnp.float32)
    m = jnp.max(logits, axis=-1, keepdims=True)
    ex = jnp.exp(logits - m)
    probs = ex / jnp.sum(ex, axis=-1, keepdims=True)

    iota = jax.lax.broadcasted_iota(jnp.int32, probs.shape, 1)
    p1 = jnp.max(probs, axis=-1, keepdims=True)
    idx1 = jnp.min(jnp.where(probs == p1, iota, E), axis=-1, keepdims=True)
    oh1 = (iota == idx1)
    masked = jnp.where(oh1, -jnp.inf, probs)
    p2 = jnp.max(masked, axis=-1, keepdims=True)
    idx2 = jnp.min(jnp.where(masked == p2, iota, E), axis=-1, keepdims=True)
    oh2 = (iota == idx2)

    denom = p1 + p2 + 1e-9
    oh1f = oh1.astype(jnp.float32)
    oh2f = oh2.astype(jnp.float32)
    w1x_ref[...] = jnp.broadcast_to(p1 / denom, (s, 128))
    w2x_ref[...] = jnp.broadcast_to(p2 / denom, (s, 128))

    sf = jnp.float32(s)
    f = jnp.sum(oh1f + oh2f, axis=0) / sf
    pmean = jnp.sum(probs, axis=0) / sf
    aux_ref[...] = jnp.reshape(LB_W * E * jnp.sum(f * pmean), (1, 1))

    O = jnp.concatenate([oh1f, oh2f], axis=0)
    a2 = 2 * s
    ri = jax.lax.broadcasted_iota(jnp.int32, (a2, a2), 0)
    ci = jax.lax.broadcasted_iota(jnp.int32, (a2, a2), 1)
    L = (ci < ri).astype(jnp.bfloat16)
    rank = jax.lax.dot_general(
        L, O.astype(jnp.bfloat16), (((1,), (0,)), ((), ())),
        preferred_element_type=jnp.float32)

    counts = jnp.sum(O, axis=0, keepdims=True)
    pcounts = jnp.ceil(counts / TT) * TT
    ce = jax.lax.broadcasted_iota(jnp.int32, (E, E), 0)
    cc = jax.lax.broadcasted_iota(jnp.int32, (E, E), 1)
    tri = (ce < cc).astype(jnp.float32)
    offs_pad = jax.lax.dot_general(
        pcounts, tri, (((1,), (0,)), ((), ())),
        preferred_element_type=jnp.float32)

    pos = jnp.sum(O * (rank + offs_pad), axis=1, keepdims=True)
    posi = pos.astype(jnp.int32)
    pos1_ref[...] = posi[:s]
    pos2_ref[...] = posi[s:]

    tstart = (jax.lax.broadcasted_iota(jnp.int32, (1, NTP), 1) * TT
              ).astype(jnp.float32)
    op_col = jnp.broadcast_to(offs_pad.reshape(E, 1), (E, NTP))
    te = jnp.sum((op_col <= tstart).astype(jnp.float32), axis=0,
                 keepdims=True) - 1.0
    seg_end = offs_pad + counts
    te_b = jnp.broadcast_to(te, (E, NTP))
    e_col = jax.lax.broadcasted_iota(jnp.int32, (E, NTP), 0).astype(jnp.float32)
    send = jnp.sum(jnp.where(te_b == e_col,
                             jnp.broadcast_to(seg_end.reshape(E, 1), (E, NTP)),
                             0.0), axis=0, keepdims=True)
    te_ref[...] = te.astype(jnp.int32)
    send_ref[...] = send.astype(jnp.int32)


def _ffn_body(te_ref, send_ref, xg_ref, ws_ref, wgu_ref, wd_ref, y_ref,
              wgub_s, wdb_s):
    n = pl.program_id(0)
    e = te_ref[n]
    seg_end = send_ref[n]
    prev_e = jnp.where(n == 0, -1, te_ref[jnp.maximum(n - 1, 0)])

    @pl.when(e != prev_e)
    def _():
        wgub_s[...] = wgu_ref[0].astype(jnp.bfloat16)
        wdb_s[...] = wd_ref[0].astype(jnp.bfloat16)

    used = seg_end > n * TT

    @pl.when(used)
    def _():
        ri = jax.lax.broadcasted_iota(jnp.int32, (TT, 1), 0) + n * TT
        rmask = ri < seg_end
        x = jnp.where(rmask, xg_ref[...], 0.0).astype(jnp.bfloat16)
        gu = jnp.dot(x, wgub_s[...], preferred_element_type=jnp.float32)
        g = gu[:, :F]
        u = gu[:, F:]
        act = (g * jax.nn.sigmoid(g) * u).astype(jnp.bfloat16)
        y = jnp.dot(act, wdb_s[...], preferred_element_type=jnp.float32)
        ws = jnp.where(rmask, ws_ref[:, 0:1], 0.0)
        y_ref[...] = y * ws

    @pl.when(jnp.logical_not(used))
    def _():
        y_ref[...] = jnp.zeros_like(y_ref)


_info = plsc.get_sparse_core_info()
_NC = _info.num_cores
_NS = _info.num_subcores
_NW = _NC * _NS


def _make_scatter(s, h):
    tpw = s // _NW
    mesh = plsc.VectorSubcoreMesh(core_axis_name="c", subcore_axis_name="s")

    @functools.partial(
        pl.kernel, mesh=mesh,
        out_type=[
            jax.ShapeDtypeStruct((PR, h), jnp.float32),
            jax.ShapeDtypeStruct((PR, 128), jnp.float32),
        ],
        scratch_types=[
            pltpu.VMEM((tpw,), jnp.int32),
            pltpu.VMEM((tpw,), jnp.int32),
            pltpu.VMEM((tpw, h), jnp.float32),
            pltpu.VMEM((tpw, 128), jnp.float32),
            pltpu.VMEM((tpw, 128), jnp.float32),
            pltpu.SemaphoreType.DMA,
        ],
    )
    def k(x_hbm, pos1_hbm, pos2_hbm, w1x_hbm, w2x_hbm, xg_hbm, ws_hbm,
          idx1_v, idx2_v, xv, w1v, w2v, sem):
        wid = lax.axis_index("s") * _NC + lax.axis_index("c")
        base = wid * tpw
        pltpu.sync_copy(pos1_hbm.at[pl.ds(base, tpw)], idx1_v)
        pltpu.sync_copy(pos2_hbm.at[pl.ds(base, tpw)], idx2_v)
        pltpu.sync_copy(x_hbm.at[pl.ds(base, tpw), :], xv)
        pltpu.sync_copy(w1x_hbm.at[pl.ds(base, tpw), :], w1v)
        pltpu.sync_copy(w2x_hbm.at[pl.ds(base, tpw), :], w2v)
        c1 = pltpu.async_copy(xv, xg_hbm.at[idx1_v], sem)
        c2 = pltpu.async_copy(xv, xg_hbm.at[idx2_v], sem)
        c3 = pltpu.async_copy(w1v, ws_hbm.at[idx1_v], sem)
        c4 = pltpu.async_copy(w2v, ws_hbm.at[idx2_v], sem)
        c1.wait()
        c2.wait()
        c3.wait()
        c4.wait()

    return k


def _make_combine(s, h):
    tpw = s // _NW
    half = tpw // 2
    mesh = plsc.VectorSubcoreMesh(core_axis_name="c", subcore_axis_name="s")

    @functools.partial(
        pl.kernel, mesh=mesh,
        out_type=jax.ShapeDtypeStruct((s, h), jnp.float32),
        scratch_types=[
            pltpu.VMEM((half,), jnp.int32),
            pltpu.VMEM((half,), jnp.int32),
            pltpu.VMEM((half, h), jnp.float32),
            pltpu.VMEM((half, h), jnp.float32),
            pltpu.SemaphoreType.DMA,
        ],
    )
    def k(y_hbm, pos1_hbm, pos2_hbm, out_hbm, i1, i2, r1, r2, sem):
        wid = lax.axis_index("s") * _NC + lax.axis_index("c")

        def chunk(ci, carry):
            base = wid * tpw + ci * half
            pltpu.sync_copy(pos1_hbm.at[pl.ds(base, half)], i1)
            pltpu.sync_copy(pos2_hbm.at[pl.ds(base, half)], i2)
            pltpu.async_copy(y_hbm.at[i1], r1, sem).wait()
            pltpu.async_copy(y_hbm.at[i2], r2, sem).wait()

            def row(r, carry2):
                for c in range(h // 16):
                    sl = pl.ds(c * 16, 16)
                    r1[r, sl] = r1[r, sl] + r2[r, sl]
                return carry2

            lax.fori_loop(0, half, row, 0)
            pltpu.sync_copy(r1, out_hbm.at[pl.ds(base, half), :])
            return carry

        lax.fori_loop(0, 2, chunk, 0)

    return k


def kernel(x, gate_w, w_gate_up, w_down):
    b, s, h = x.shape
    x_flat = x.reshape(s, h)

    pos1, pos2, w1x, w2x, te, send, aux = pl.pallas_call(
        _gate_body,
        out_shape=[
            jax.ShapeDtypeStruct((s, 1), jnp.int32),
            jax.ShapeDtypeStruct((s, 1), jnp.int32),
            jax.ShapeDtypeStruct((s, 128), jnp.float32),
            jax.ShapeDtypeStruct((s, 128), jnp.float32),
            jax.ShapeDtypeStruct((1, NTP), jnp.int32),
            jax.ShapeDtypeStruct((1, NTP), jnp.int32),
            jax.ShapeDtypeStruct((1, 1), jnp.float32),
        ],
        compiler_params=pltpu.CompilerParams(
            vmem_limit_bytes=120 * 1024 * 1024,
        ),
    )(x_flat, gate_w)

    p1 = pos1.reshape(s)
    p2 = pos2.reshape(s)
    xg, wsort = _make_scatter(s, h)(x_flat, p1, p2, w1x, w2x)

    grid_spec = pltpu.PrefetchScalarGridSpec(
        num_scalar_prefetch=2,
        grid=(NTP,),
        in_specs=[
            pl.BlockSpec((TT, H), lambda n, te, send: (n, 0)),
            pl.BlockSpec((TT, 128), lambda n, te, send: (n, 0)),
            pl.BlockSpec((1, H, 2 * F), lambda n, te, send: (te[n], 0, 0)),
            pl.BlockSpec((1, F, H), lambda n, te, send: (te[n], 0, 0)),
        ],
        out_specs=pl.BlockSpec((TT, H), lambda n, te, send: (n, 0)),
        scratch_shapes=[
            pltpu.VMEM((H, 2 * F), jnp.bfloat16),
            pltpu.VMEM((F, H), jnp.bfloat16),
        ],
    )
    y = pl.pallas_call(
        _ffn_body,
        grid_spec=grid_spec,
        out_shape=jax.ShapeDtypeStruct((PR, H), jnp.float32),
        compiler_params=pltpu.CompilerParams(
            vmem_limit_bytes=120 * 1024 * 1024,
        ),
    )(te.reshape(NTP), send.reshape(NTP), xg, wsort, w_gate_up, w_down)

    out = _make_combine(s, h)(y, p1, p2)
    return out.reshape(b, s, h), aux[0, 0]


# speedup vs baseline: 1.0643x; 1.0611x over previous
import functools

import jax
import jax.numpy as jnp
from jax import lax
from jax.experimental import pallas as pl
from jax.experimental.pallas import tpu as pltpu
from jax.experimental.pallas import tpu_sc as plsc

E = 8
H = 768
F = 2048
TT = 768
NTP = 14
PR = NTP * TT
LB_W = 0.01


def _gate_body(x_ref, gw_ref, pos1_ref, pos2_ref, w1x_ref, w2x_ref,
               te_ref, send_ref, aux_ref):
    x = x_ref[...]
    gw = gw_ref[...]
    s = x.shape[0]
    logits = jax.lax.dot_general(
        x, gw, (((1,), (1,)), ((), ())), preferred_element_type=jnp.float32)
    m = jnp.max(logits, axis=-1, keepdims=True)
    ex = jnp.exp(logits - m)
    probs = ex / jnp.sum(ex, axis=-1, keepdims=True)

    iota = jax.lax.broadcasted_iota(jnp.int32, probs.shape, 1)
    p1 = jnp.max(probs, axis=-1, keepdims=True)
    idx1 = jnp.min(jnp.where(probs == p1, iota, E), axis=-1, keepdims=True)
    oh1 = (iota == idx1)
    masked = jnp.where(oh1, -jnp.inf, probs)
    p2 = jnp.max(masked, axis=-1, keepdims=True)
    idx2 = jnp.min(jnp.where(masked == p2, iota, E), axis=-1, keepdims=True)
    oh2 = (iota == idx2)

    denom = p1 + p2 + 1e-9
    oh1f = oh1.astype(jnp.float32)
    oh2f = oh2.astype(jnp.float32)
    w1x_ref[...] = jnp.broadcast_to(p1 / denom, (s, 128))
    w2x_ref[...] = jnp.broadcast_to(p2 / denom, (s, 128))

    sf = jnp.float32(s)
    f = jnp.sum(oh1f + oh2f, axis=0) / sf
    pmean = jnp.sum(probs, axis=0) / sf
    aux_ref[...] = jnp.reshape(LB_W * E * jnp.sum(f * pmean), (1, 1))

    O = jnp.concatenate([oh1f, oh2f], axis=0)
    a2 = 2 * s
    CH = 512
    ri = jax.lax.broadcasted_iota(jnp.int32, (CH, CH), 0)
    ci = jax.lax.broadcasted_iota(jnp.int32, (CH, CH), 1)
    L = (ci < ri).astype(jnp.bfloat16)
    running = jnp.zeros((1, E), jnp.float32)
    parts = []
    for c in range(a2 // CH):
        oc = O[c * CH:(c + 1) * CH]
        rc = jax.lax.dot_general(
            L, oc.astype(jnp.bfloat16), (((1,), (0,)), ((), ())),
            preferred_element_type=jnp.float32)
        parts.append(rc + running)
        running = running + jnp.sum(oc, axis=0, keepdims=True)
    rank = jnp.concatenate(parts, axis=0)

    counts = jnp.sum(O, axis=0, keepdims=True)
    pcounts = jnp.ceil(counts / TT) * TT
    ce = jax.lax.broadcasted_iota(jnp.int32, (E, E), 0)
    cc = jax.lax.broadcasted_iota(jnp.int32, (E, E), 1)
    tri = (ce < cc).astype(jnp.float32)
    offs_pad = jax.lax.dot_general(
        pcounts, tri, (((1,), (0,)), ((), ())),
        preferred_element_type=jnp.float32)

    pos = jnp.sum(O * (rank + offs_pad), axis=1, keepdims=True)
    posi = pos.astype(jnp.int32)
    pos1_ref[...] = posi[:s]
    pos2_ref[...] = posi[s:]

    tstart = (jax.lax.broadcasted_iota(jnp.int32, (1, NTP), 1) * TT
              ).astype(jnp.float32)
    op_col = jnp.broadcast_to(offs_pad.reshape(E, 1), (E, NTP))
    te = jnp.sum((op_col <= tstart).astype(jnp.float32), axis=0,
                 keepdims=True) - 1.0
    seg_end = offs_pad + counts
    te_b = jnp.broadcast_to(te, (E, NTP))
    e_col = jax.lax.broadcasted_iota(jnp.int32, (E, NTP), 0).astype(jnp.float32)
    send = jnp.sum(jnp.where(te_b == e_col,
                             jnp.broadcast_to(seg_end.reshape(E, 1), (E, NTP)),
                             0.0), axis=0, keepdims=True)
    te_ref[...] = te.astype(jnp.int32)
    send_ref[...] = send.astype(jnp.int32)


def _ffn_body(te_ref, send_ref, xg_ref, ws_ref, wgu_ref, wd_ref, y_ref,
              wgub_s, wdb_s):
    n = pl.program_id(0)
    e = te_ref[n]
    seg_end = send_ref[n]
    prev_e = jnp.where(n == 0, -1, te_ref[jnp.maximum(n - 1, 0)])

    @pl.when(e != prev_e)
    def _():
        wgub_s[...] = wgu_ref[0].astype(jnp.bfloat16)
        wdb_s[...] = wd_ref[0].astype(jnp.bfloat16)

    used = seg_end > n * TT

    @pl.when(used)
    def _():
        ri = jax.lax.broadcasted_iota(jnp.int32, (TT, 1), 0) + n * TT
        rmask = ri < seg_end
        x = jnp.where(rmask, xg_ref[...], 0.0).astype(jnp.bfloat16)
        gu = jnp.dot(x, wgub_s[...], preferred_element_type=jnp.float32)
        g = gu[:, :F]
        u = gu[:, F:]
        act = (g * jax.nn.sigmoid(g) * u).astype(jnp.bfloat16)
        y = jnp.dot(act, wdb_s[...], preferred_element_type=jnp.float32)
        ws = jnp.where(rmask, ws_ref[:, 0:1], 0.0)
        y_ref[...] = y * ws

    @pl.when(jnp.logical_not(used))
    def _():
        y_ref[...] = jnp.zeros_like(y_ref)


_info = plsc.get_sparse_core_info()
_NC = _info.num_cores
_NS = _info.num_subcores
_NW = _NC * _NS


def _make_scatter(s, h):
    tpw = s // _NW
    mesh = plsc.VectorSubcoreMesh(core_axis_name="c", subcore_axis_name="s")

    @functools.partial(
        pl.kernel, mesh=mesh,
        out_type=[
            jax.ShapeDtypeStruct((PR, h), jnp.float32),
            jax.ShapeDtypeStruct((PR, 128), jnp.float32),
        ],
        scratch_types=[
            pltpu.VMEM((tpw,), jnp.int32),
            pltpu.VMEM((tpw,), jnp.int32),
            pltpu.VMEM((tpw, h), jnp.float32),
            pltpu.VMEM((tpw, 128), jnp.float32),
            pltpu.VMEM((tpw, 128), jnp.float32),
            pltpu.SemaphoreType.DMA,
        ],
    )
    def k(x_hbm, pos1_hbm, pos2_hbm, w1x_hbm, w2x_hbm, xg_hbm, ws_hbm,
          idx1_v, idx2_v, xv, w1v, w2v, sem):
        wid = lax.axis_index("s") * _NC + lax.axis_index("c")
        base = wid * tpw
        pltpu.sync_copy(pos1_hbm.at[pl.ds(base, tpw)], idx1_v)
        pltpu.sync_copy(pos2_hbm.at[pl.ds(base, tpw)], idx2_v)
        pltpu.sync_copy(x_hbm.at[pl.ds(base, tpw), :], xv)
        pltpu.sync_copy(w1x_hbm.at[pl.ds(base, tpw), :], w1v)
        pltpu.sync_copy(w2x_hbm.at[pl.ds(base, tpw), :], w2v)
        c1 = pltpu.async_copy(xv, xg_hbm.at[idx1_v], sem)
        c2 = pltpu.async_copy(xv, xg_hbm.at[idx2_v], sem)
        c3 = pltpu.async_copy(w1v, ws_hbm.at[idx1_v], sem)
        c4 = pltpu.async_copy(w2v, ws_hbm.at[idx2_v], sem)
        c1.wait()
        c2.wait()
        c3.wait()
        c4.wait()

    return k


def _make_combine(s, h):
    tpw = s // _NW
    half = tpw // 2
    mesh = plsc.VectorSubcoreMesh(core_axis_name="c", subcore_axis_name="s")

    @functools.partial(
        pl.kernel, mesh=mesh,
        out_type=jax.ShapeDtypeStruct((s, h), jnp.float32),
        scratch_types=[
            pltpu.VMEM((half,), jnp.int32),
            pltpu.VMEM((half,), jnp.int32),
            pltpu.VMEM((half, h), jnp.float32),
            pltpu.VMEM((half, h), jnp.float32),
            pltpu.SemaphoreType.DMA,
        ],
    )
    def k(y_hbm, pos1_hbm, pos2_hbm, out_hbm, i1, i2, r1, r2, sem):
        wid = lax.axis_index("s") * _NC + lax.axis_index("c")

        def chunk(ci, carry):
            base = wid * tpw + ci * half
            pltpu.sync_copy(pos1_hbm.at[pl.ds(base, half)], i1)
            pltpu.sync_copy(pos2_hbm.at[pl.ds(base, half)], i2)
            pltpu.async_copy(y_hbm.at[i1], r1, sem).wait()
            pltpu.async_copy(y_hbm.at[i2], r2, sem).wait()

            def row(r, carry2):
                for c in range(h // 16):
                    sl = pl.ds(c * 16, 16)
                    r1[r, sl] = r1[r, sl] + r2[r, sl]
                return carry2

            lax.fori_loop(0, half, row, 0)
            pltpu.sync_copy(r1, out_hbm.at[pl.ds(base, half), :])
            return carry

        lax.fori_loop(0, 2, chunk, 0)

    return k


def kernel(x, gate_w, w_gate_up, w_down):
    b, s, h = x.shape
    x_flat = x.reshape(s, h)

    pos1, pos2, w1x, w2x, te, send, aux = pl.pallas_call(
        _gate_body,
        out_shape=[
            jax.ShapeDtypeStruct((s, 1), jnp.int32),
            jax.ShapeDtypeStruct((s, 1), jnp.int32),
            jax.ShapeDtypeStruct((s, 128), jnp.float32),
            jax.ShapeDtypeStruct((s, 128), jnp.float32),
            jax.ShapeDtypeStruct((1, NTP), jnp.int32),
            jax.ShapeDtypeStruct((1, NTP), jnp.int32),
            jax.ShapeDtypeStruct((1, 1), jnp.float32),
        ],
        compiler_params=pltpu.CompilerParams(
            vmem_limit_bytes=120 * 1024 * 1024,
        ),
    )(x_flat, gate_w)

    p1 = pos1.reshape(s)
    p2 = pos2.reshape(s)
    xg, wsort = _make_scatter(s, h)(x_flat, p1, p2, w1x, w2x)

    grid_spec = pltpu.PrefetchScalarGridSpec(
        num_scalar_prefetch=2,
        grid=(NTP,),
        in_specs=[
            pl.BlockSpec((TT, H), lambda n, te, send: (n, 0)),
            pl.BlockSpec((TT, 128), lambda n, te, send: (n, 0)),
            pl.BlockSpec((1, H, 2 * F), lambda n, te, send: (te[n], 0, 0)),
            pl.BlockSpec((1, F, H), lambda n, te, send: (te[n], 0, 0)),
        ],
        out_specs=pl.BlockSpec((TT, H), lambda n, te, send: (n, 0)),
        scratch_shapes=[
            pltpu.VMEM((H, 2 * F), jnp.bfloat16),
            pltpu.VMEM((F, H), jnp.bfloat16),
        ],
    )
    y = pl.pallas_call(
        _ffn_body,
        grid_spec=grid_spec,
        out_shape=jax.ShapeDtypeStruct((PR, H), jnp.float32),
        compiler_params=pltpu.CompilerParams(
            vmem_limit_bytes=120 * 1024 * 1024,
        ),
    )(te.reshape(NTP), send.reshape(NTP), xg, wsort, w_gate_up, w_down)

    out = _make_combine(s, h)(y, p1, p2)
    return out.reshape(b, s, h), aux[0, 0]
